# baseline (speedup 1.0000x reference)
import functools

import jax
import jax.numpy as jnp
from jax import lax
from jax.experimental import pallas as pl
from jax.experimental.pallas import tpu as pltpu
from jax.experimental.pallas import tpu_sc as plsc

BATCH = 16384
NA = 8

_info = plsc.get_sparse_core_info()
_NC, _NS, _L = _info.num_cores, _info.num_subcores, _info.num_lanes
_NW = _NC * _NS
_BPW = BATCH // _NW
_GCHUNK = 128
_NCHUNK = _BPW // _GCHUNK


def _body(a0_hbm, a1_hbm, s_hbm, table_hbm, out_hbm,
          a0_v, a1_v, s_v, idx_v, val_v, sem):
    wid = lax.axis_index("s") * _NC + lax.axis_index("c")
    base = wid * _BPW

    pltpu.sync_copy(a0_hbm.at[pl.ds(base, _BPW)], a0_v)
    pltpu.sync_copy(a1_hbm.at[pl.ds(base, _BPW)], a1_v)
    pltpu.sync_copy(s_hbm.at[pl.ds(base, _BPW)], s_v)

    for i in range(_BPW // _L):
        sl = pl.ds(i * _L, _L)
        flat = s_v[sl] * (NA * NA) + a0_v[sl] * NA + a1_v[sl]
        idx_v[i // (_GCHUNK // _L), pl.ds((i % (_GCHUNK // _L)) * _L, _L)] = flat

    copies = [
        pltpu.async_copy(table_hbm.at[idx_v.at[j]],
                         val_v.at[pl.ds(j * _GCHUNK, _GCHUNK)], sem)
        for j in range(_NCHUNK)
    ]
    for c in copies:
        c.wait()

    for i in range(_BPW // _L):
        sl = pl.ds(i * _L, _L)
        x = val_v[sl]
        val_v[sl] = 1.0 / (1.0 + jnp.exp(-x))

    pltpu.sync_copy(val_v, out_hbm.at[pl.ds(base, _BPW)])


@functools.partial(jax.jit, static_argnames=())
def kernel(a0, a1, s_idx, logits):
    flat_table = logits.reshape(-1)
    a0 = a0.astype(jnp.int32)
    a1 = a1.astype(jnp.int32)
    s_idx = s_idx.astype(jnp.int32)
    run = pl.kernel(
        _body,
        mesh=plsc.VectorSubcoreMesh(core_axis_name="c", subcore_axis_name="s"),
        out_type=jax.ShapeDtypeStruct((BATCH,), jnp.float32),
        scratch_types=[
            pltpu.VMEM((_BPW,), jnp.int32),
            pltpu.VMEM((_BPW,), jnp.int32),
            pltpu.VMEM((_BPW,), jnp.int32),
            pltpu.VMEM((_NCHUNK, _GCHUNK), jnp.int32),
            pltpu.VMEM((_BPW,), jnp.float32),
            pltpu.SemaphoreType.DMA,
        ],
    )
    return run(a0, a1, s_idx, flat_table)


# speedup vs baseline: 99.1436x; 99.1436x over previous
import functools

import jax
import jax.numpy as jnp
from jax import lax
from jax.experimental import pallas as pl
from jax.experimental.pallas import tpu as pltpu
from jax.experimental.pallas import tpu_sc as plsc

NUM_STATES = 1000000
BATCH = 16384
NA = 8

_info = plsc.get_sparse_core_info()
_NC, _NS, _L = _info.num_cores, _info.num_subcores, _info.num_lanes
_NW = _NC * _NS
_BPW = BATCH // _NW
_K = 64
_NCH = _BPW // _K


def _body(a0_hbm, a1_hbm, s_hbm, table_hbm, out_hbm,
          a0_v, a1_v, s_v, pos_v, tb0, tb1, val_v, sem0, sem1):
    wid = lax.axis_index("s") * _NC + lax.axis_index("c")
    base = wid * _BPW

    pltpu.sync_copy(a0_hbm.at[pl.ds(base, _BPW)], a0_v)
    pltpu.sync_copy(a1_hbm.at[pl.ds(base, _BPW)], a1_v)
    pltpu.sync_copy(s_hbm.at[pl.ds(base, _BPW)], s_v)

    for i in range(_BPW // _L):
        sl = pl.ds(i * _L, _L)
        s = s_v[sl]
        pos_v[sl] = (((lax.iota(jnp.int32, _L) + i * _L) & (_K - 1)) << 7) | (s & 127)
        a0_v[sl] = (a0_v[sl] << 19) | (a1_v[sl] << 16) | (s >> 7)

    tbs = (tb0, tb1)
    sems = (sem0, sem1)

    def fire(c, buf):
        def one(q, carry):
            p16 = a0_v[pl.ds(pl.multiple_of(c * _K + q * _L, _L), _L)]
            for j in range(_L):
                p = p16[j]
                tile = table_hbm.at[
                    p >> 19, :,
                    pl.ds(pl.multiple_of((p & 8191) << 7, 128), 128)
                ]
                pltpu.async_copy(
                    tile.at[(p >> 16) & 7],
                    tbs[buf].at[pl.ds(
                        pl.multiple_of((q * _L + j) * 128, 128), 128)],
                    sems[buf],
                )
            return carry
        lax.fori_loop(0, _K // _L, one, 0)

    def drain(buf):
        pltpu.make_async_copy(
            out_hbm.at[pl.ds(0, _K * 128)], tbs[buf], sems[buf]
        ).wait()

    def pick(c, buf):
        for i in range(_K // _L):
            sl = pl.ds(c * _K + i * _L, _L)
            x = plsc.load_gather(tbs[buf], [pos_v[sl]])
            val_v[sl] = 1.0 / (1.0 + jnp.exp(-x))

    fire(0, 0)
    for c in range(_NCH):
        if c + 1 < _NCH:
            fire(c + 1, (c + 1) % 2)
        drain(c % 2)
        pick(c, c % 2)

    pltpu.sync_copy(val_v, out_hbm.at[pl.ds(base, _BPW)])


@functools.partial(jax.jit, static_argnames=())
def kernel(a0, a1, s_idx, logits):
    tab = jnp.transpose(logits, (1, 2, 0))
    a0 = a0.astype(jnp.int32)
    a1 = a1.astype(jnp.int32)
    s_idx = s_idx.astype(jnp.int32)
    run = pl.kernel(
        _body,
        mesh=plsc.VectorSubcoreMesh(core_axis_name="c", subcore_axis_name="s"),
        out_type=jax.ShapeDtypeStruct((BATCH,), jnp.float32),
        compiler_params=pltpu.CompilerParams(needs_layout_passes=False),
        scratch_types=[
            pltpu.VMEM((_BPW,), jnp.int32),
            pltpu.VMEM((_BPW,), jnp.int32),
            pltpu.VMEM((_BPW,), jnp.int32),
            pltpu.VMEM((_BPW,), jnp.int32),
            pltpu.VMEM((_K * 128,), jnp.float32),
            pltpu.VMEM((_K * 128,), jnp.float32),
            pltpu.VMEM((_BPW,), jnp.float32),
            pltpu.SemaphoreType.DMA,
            pltpu.SemaphoreType.DMA,
        ],
    )
    return run(a0, a1, s_idx, tab)
